# baseline (speedup 1.0000x reference)
import functools

import jax
import jax.numpy as jnp
from jax import lax
from jax.experimental import pallas as pl
from jax.experimental.pallas import tpu as pltpu
from jax.experimental.pallas import tpu_sc as plsc


def _make_sc_gather(V, D, N):
    info = plsc.get_sparse_core_info()
    NC, NS = info.num_cores, info.num_subcores
    NW = NC * NS

    SEG = 1024
    K = 1
    C = K * SEG
    per_w = N // NW
    n_chunks = per_w // C
    assert per_w % C == 0 and N % NW == 0

    mesh = plsc.VectorSubcoreMesh(core_axis_name="c", subcore_axis_name="s")

    @functools.partial(
        pl.kernel,
        mesh=mesh,
        out_type=jax.ShapeDtypeStruct((N, D), jnp.float32),
        scratch_types=[
            pltpu.VMEM((K, SEG), jnp.int32),
            pltpu.VMEM((C, D), jnp.float32),
            pltpu.SemaphoreType.DMA,
        ],
        compiler_params=pltpu.CompilerParams(use_tc_tiling_on_sc=False),
    )
    def grab(idx_hbm, table_hbm, out_hbm, idx_v, rows_v, sem):
        wid = lax.axis_index("s") * NC + lax.axis_index("c")

        def chunk_body(c, _):
            blk = (wid * n_chunks + c) * K
            pltpu.sync_copy(idx_hbm.at[pl.ds(blk, K)], idx_v)
            copies = []
            for j in range(K):
                copies.append(
                    pltpu.async_copy(
                        table_hbm.at[idx_v.at[j]],
                        rows_v.at[pl.ds(j * SEG, SEG)],
                        sem,
                    )
                )
            for cp in copies:
                cp.wait()
            base = (wid * n_chunks + c) * C
            pltpu.sync_copy(rows_v, out_hbm.at[pl.ds(base, C)])
            return ()

        lax.fori_loop(0, n_chunks, chunk_body, ())

    return grab


def kernel(input_seq, table):
    B, H = input_seq.shape
    V, D = table.shape
    N = B * H
    idx2d = input_seq.reshape(N // 1024, 1024).astype(jnp.int32)
    grab = _make_sc_gather(V, D, N)
    out = grab(idx2d, table)
    return out.reshape(B, H, D)


# speedup vs baseline: 1.0496x; 1.0496x over previous
import functools

import jax
import jax.numpy as jnp
from jax import lax
from jax.experimental import pallas as pl
from jax.experimental.pallas import tpu as pltpu
from jax.experimental.pallas import tpu_sc as plsc

_NBUF = 4
_C = 512


def _make_sc_gather(V, D, N):
    info = plsc.get_sparse_core_info()
    NC, NS = info.num_cores, info.num_subcores
    NW = NC * NS

    C = _C
    per_w = N // NW
    n_chunks = per_w // C
    assert per_w % C == 0 and N % NW == 0 and n_chunks > 2 * _NBUF

    mesh = plsc.VectorSubcoreMesh(core_axis_name="c", subcore_axis_name="s")

    @functools.partial(
        pl.kernel,
        mesh=mesh,
        out_type=jax.ShapeDtypeStruct((N, D), jnp.float32),
        scratch_types=[
            pltpu.VMEM((_NBUF, C), jnp.int32),
            pltpu.VMEM((_NBUF, C, D), jnp.float32),
        ]
        + [pltpu.SemaphoreType.DMA] * (2 * _NBUF),
        compiler_params=pltpu.CompilerParams(use_tc_tiling_on_sc=False),
    )
    def grab(idx_hbm, table_hbm, out_hbm, idx_v, rows_v, *sems):
        gsem = sems[:_NBUF]
        ssem = sems[_NBUF:]
        wid = lax.axis_index("s") * NC + lax.axis_index("c")
        w_chunk0 = wid * n_chunks

        def fire_gather(c, b):
            pltpu.sync_copy(idx_hbm.at[w_chunk0 + c], idx_v.at[b])
            pltpu.async_copy(table_hbm.at[idx_v.at[b]], rows_v.at[b], gsem[b])

        def wait_gather(b):
            pltpu.make_async_copy(
                table_hbm.at[idx_v.at[b]], rows_v.at[b], gsem[b]
            ).wait()

        def fire_store(c, b):
            base = (w_chunk0 + c) * C
            pltpu.async_copy(rows_v.at[b], out_hbm.at[pl.ds(base, C)], ssem[b])

        def wait_store(b):
            pltpu.make_async_copy(
                rows_v.at[b], out_hbm.at[pl.ds(0, C)], ssem[b]
            ).wait()

        for b in range(_NBUF - 1):
            fire_gather(b, b)

        def tick(c0, _):
            for b in range(_NBUF):
                c = c0 + b
                wait_gather(b)
                fire_store(c, b)
                t = (b + _NBUF - 1) % _NBUF

                @pl.when(c + _NBUF - 1 < n_chunks)
                def _():
                    @pl.when(c >= 1)
                    def _():
                        wait_store(t)

                    fire_gather(c + _NBUF - 1, t)

            return ()

        lax.fori_loop(0, n_chunks // _NBUF, lambda i, _: tick(i * _NBUF, _), ())

        for b in range(_NBUF):
            wait_store(b)

    return grab


def kernel(input_seq, table):
    B, H = input_seq.shape
    V, D = table.shape
    N = B * H
    idx2d = input_seq.reshape(N // _C, _C).astype(jnp.int32)
    grab = _make_sc_gather(V, D, N)
    out = grab(idx2d, table)
    return out.reshape(B, H, D)


# speedup vs baseline: 1.0509x; 1.0012x over previous
import functools

import jax
import jax.numpy as jnp
from jax import lax
from jax.experimental import pallas as pl
from jax.experimental.pallas import tpu as pltpu
from jax.experimental.pallas import tpu_sc as plsc

_NBUF = 4
_R = 4


def _make_sc_gather(V, D, B, H):
    info = plsc.get_sparse_core_info()
    NC, NS = info.num_cores, info.num_subcores
    NW = NC * NS

    rows_per_w = B // NW
    n_chunks = rows_per_w // _R
    assert B % NW == 0 and rows_per_w % _R == 0 and n_chunks > 2 * _NBUF

    mesh = plsc.VectorSubcoreMesh(core_axis_name="c", subcore_axis_name="s")

    @functools.partial(
        pl.kernel,
        mesh=mesh,
        out_type=jax.ShapeDtypeStruct((B, H, D), jnp.float32),
        scratch_types=[
            pltpu.VMEM((_NBUF, _R, H), jnp.int32),
            pltpu.VMEM((_NBUF, _R, H, D), jnp.float32),
        ]
        + [pltpu.SemaphoreType.DMA] * (2 * _NBUF),
        compiler_params=pltpu.CompilerParams(use_tc_tiling_on_sc=False),
    )
    def grab(idx_hbm, table_hbm, out_hbm, idx_v, rows_v, *sems):
        gsem = sems[:_NBUF]
        ssem = sems[_NBUF:]
        wid = lax.axis_index("s") * NC + lax.axis_index("c")
        w_row0 = wid * rows_per_w

        def fire_gather(c, b):
            r0 = w_row0 + c * _R
            pltpu.sync_copy(idx_hbm.at[pl.ds(r0, _R)], idx_v.at[b])
            for j in range(_R):
                pltpu.async_copy(
                    table_hbm.at[idx_v.at[b].at[j]],
                    rows_v.at[b].at[j],
                    gsem[b],
                )

        def wait_gather(b):
            for j in range(_R):
                pltpu.make_async_copy(
                    table_hbm.at[idx_v.at[b].at[j]], rows_v.at[b].at[j], gsem[b]
                ).wait()

        def fire_store(c, b):
            r0 = w_row0 + c * _R
            pltpu.async_copy(rows_v.at[b], out_hbm.at[pl.ds(r0, _R)], ssem[b])

        def wait_store(b):
            pltpu.make_async_copy(
                rows_v.at[b], out_hbm.at[pl.ds(0, _R)], ssem[b]
            ).wait()

        for b in range(_NBUF - 1):
            fire_gather(b, b)

        def tick(c0, _):
            for b in range(_NBUF):
                c = c0 + b
                wait_gather(b)
                fire_store(c, b)
                t = (b + _NBUF - 1) % _NBUF

                @pl.when(c + _NBUF - 1 < n_chunks)
                def _():
                    @pl.when(c >= 1)
                    def _():
                        wait_store(t)

                    fire_gather(c + _NBUF - 1, t)

            return ()

        lax.fori_loop(0, n_chunks // _NBUF, lambda i, _: tick(i * _NBUF, _), ())

        for b in range(_NBUF):
            wait_store(b)

    return grab


def kernel(input_seq, table):
    B, H = input_seq.shape
    V, D = table.shape
    grab = _make_sc_gather(V, D, B, H)
    return grab(input_seq.astype(jnp.int32), table)
